# baseline (speedup 1.0000x reference)
import functools

import jax
import jax.numpy as jnp
from jax import lax
from jax.experimental import pallas as pl
from jax.experimental.pallas import tpu as pltpu
from jax.experimental.pallas import tpu_sc as plsc

VS = 0.4
ORG = (0.0, -25.6, -2.0)
IMH, IMW = 370, 1220
DX, DY, DZ = 128, 128, 16
NVOX = DX * DY * DZ
B = 128


def _f2i_floor(x):
    t = x.astype(jnp.int32)
    return t - jnp.where(t.astype(jnp.float32) > x, 1, 0)


def _sanitize(x):
    x = jnp.where(x != x, 0.0, x)
    return jnp.clip(x, -2.0e9, 2.0e9)


def _bf16r(x):
    y = lax.bitcast_convert_type(x, jnp.uint32)
    r = (y + jnp.uint32(0x7FFF) + ((y >> 16) & jnp.uint32(1))) & jnp.uint32(0xFFFF0000)
    return lax.bitcast_convert_type(r, jnp.float32)


def _pool_sc(tab, dflat, projp, hw):
    H, W = hw
    C = tab.shape[1]
    n_d = dflat.shape[0]
    mesh = plsc.VectorSubcoreMesh(core_axis_name="c", subcore_axis_name="s")
    info = plsc.get_sparse_core_info()
    nw = info.num_cores * info.num_subcores
    per_w = NVOX // nw
    n_chunks = per_w // B

    @functools.partial(
        pl.kernel,
        mesh=mesh,
        out_type=jax.ShapeDtypeStruct((NVOX, C), jnp.float32),
        scratch_types=[
            pltpu.VMEM((n_d,), jnp.float32),
            pltpu.VMEM((16,), jnp.float32),
            pltpu.VMEM((B,), jnp.int32),
            pltpu.VMEM((B,), jnp.int32),
            pltpu.VMEM((B,), jnp.int32),
            pltpu.VMEM((B,), jnp.int32),
            pltpu.VMEM((B,), jnp.float32),
            pltpu.VMEM((B,), jnp.float32),
            pltpu.VMEM((B,), jnp.float32),
            pltpu.VMEM((B,), jnp.float32),
            pltpu.VMEM((B, C), jnp.float32),
            pltpu.VMEM((B, C), jnp.float32),
            pltpu.VMEM((B, C), jnp.float32),
            pltpu.VMEM((B, C), jnp.float32),
            pltpu.VMEM((B, C), jnp.float32),
            pltpu.SemaphoreType.DMA,
        ],
        compiler_params=pltpu.CompilerParams(needs_layout_passes=False),
    )
    def pool(tab_h, dep_h, proj_h, out_h, dep_v, proj_v,
             i00_v, i01_v, i10_v, i11_v, w00_v, w01_v, w10_v, w11_v,
             r00_v, r01_v, r10_v, r11_v, out_v, sem):
        wid = lax.axis_index("s") * info.num_cores + lax.axis_index("c")
        pltpu.sync_copy(dep_h, dep_v)
        pltpu.sync_copy(proj_h, proj_v)
        pvec = _bf16r(proj_v[...])
        p = [pvec[c] for c in range(12)]

        wspan = jnp.float32(W - 1)
        hspan = jnp.float32(H - 1)

        def geom(g, base):
            nvec = base + g * 16 + lax.iota(jnp.int32, 16)
            fi = (nvec >> 11).astype(jnp.float32)
            fj = ((nvec >> 4) & (DY - 1)).astype(jnp.float32)
            fk = (nvec & (DZ - 1)).astype(jnp.float32)
            fx = _bf16r(fi * VS + (0.5 * VS + ORG[0]))
            fy = _bf16r(fj * VS + (0.5 * VS + ORG[1]))
            fz = _bf16r(fk * VS + (0.5 * VS + ORG[2]))
            ix_ = ((p[0] * fx + p[1] * fy) + p[2] * fz) + p[3]
            iy_ = ((p[4] * fx + p[5] * fy) + p[6] * fz) + p[7]
            iz_ = ((p[8] * fx + p[9] * fy) + p[10] * fz) + p[11]
            u = ix_ / iz_
            v = iy_ / iz_
            gx = (2.0 * u) / jnp.float32(IMW - 1) - 1.0
            gy = (2.0 * v) / jnp.float32(IMH - 1) - 1.0
            big = jnp.float32(3.4028235e38)
            gx = jnp.where(jnp.abs(gx) > big, -2.0, gx)
            gy = jnp.where(jnp.abs(gy) > big, -2.0, gy)
            mask = (jnp.abs(gx) <= 1.0) & (jnp.abs(gy) <= 1.0) & (iz_ > 0.0)

            ixn = _f2i_floor(_sanitize(((gx + 1.0) * W - 1.0) / 2.0 + 0.5))
            iyn = _f2i_floor(_sanitize(((gy + 1.0) * H - 1.0) / 2.0 + 0.5))
            dvalid = (ixn >= 0) & (ixn < W) & (iyn >= 0) & (iyn < H)
            didx = jnp.clip(iyn, 0, H - 1) * W + jnp.clip(ixn, 0, W - 1)
            dep = plsc.load_gather(dep_v, [didx])
            dep = jnp.where(dvalid, dep, 0.0)
            diff = iz_ - dep
            prob = jnp.exp(-(diff * diff) / jnp.float32((2 * VS) ** 2))
            scale = jnp.where(mask, prob * 100.0, 0.0)

            def refl(g_, span):
                x = jnp.abs(_sanitize((g_ + 1.0) / 2.0 * span))
                x = lax.rem(x, 2.0 * span)
                return jnp.where(x > span, 2.0 * span - x, x)

            fxc = refl(gx, wspan)
            fyc = refl(gy, hspan)
            x0 = _f2i_floor(fxc)
            y0 = _f2i_floor(fyc)
            wx1 = fxc - x0.astype(jnp.float32)
            wy1 = fyc - y0.astype(jnp.float32)
            x0i = jnp.clip(x0, 0, W - 1)
            x1i = jnp.minimum(x0i + 1, W - 1)
            y0i = jnp.clip(y0, 0, H - 1)
            y1i = jnp.minimum(y0i + 1, H - 1)
            wx0 = 1.0 - wx1
            wy0 = 1.0 - wy1
            sl = pl.ds(g * 16, 16)
            i00_v[sl] = y0i * W + x0i
            i01_v[sl] = y0i * W + x1i
            i10_v[sl] = y1i * W + x0i
            i11_v[sl] = y1i * W + x1i
            w00_v[sl] = wy0 * wx0 * scale
            w01_v[sl] = wy0 * wx1 * scale
            w10_v[sl] = wy1 * wx0 * scale
            w11_v[sl] = wy1 * wx1 * scale
            return base

        def blend(g, _):
            sl = pl.ds(g * 16, 16)
            wa = w00_v[sl]
            wb = w01_v[sl]
            wc = w10_v[sl]
            wd = w11_v[sl]
            for li in range(16):
                vv = g * 16 + li
                a = wa[li]
                b = wb[li]
                c = wc[li]
                dd = wd[li]
                for cc in range(C // 16):
                    s = pl.ds(cc * 16, 16)
                    out_v[vv, s] = (a * r00_v[vv, s] + b * r01_v[vv, s]
                                    + c * r10_v[vv, s] + dd * r11_v[vv, s])
            return 0

        def chunk(t, _):
            base = wid * per_w + t * B
            lax.fori_loop(0, B // 16, geom, base, unroll=2)
            cp0 = pltpu.async_copy(tab_h.at[i00_v], r00_v, sem)
            cp1 = pltpu.async_copy(tab_h.at[i01_v], r01_v, sem)
            cp2 = pltpu.async_copy(tab_h.at[i10_v], r10_v, sem)
            cp3 = pltpu.async_copy(tab_h.at[i11_v], r11_v, sem)
            cp0.wait()
            cp1.wait()
            cp2.wait()
            cp3.wait()
            lax.fori_loop(0, B // 16, blend, 0)
            pltpu.sync_copy(out_v, out_h.at[pl.ds(base, B)])
            return 0

        lax.fori_loop(0, n_chunks, chunk, 0)

    return pool(tab, dflat, projp)


def _ln(x, g, b):
    m = jnp.mean(x, -1, keepdims=True)
    v = jnp.mean((x - m) ** 2, -1, keepdims=True)
    return (x - m) / jnp.sqrt(v + 1e-5) * g + b


def _leaky(x):
    return jnp.where(x >= 0, x, 0.1 * x)


def _bdot(a, b):
    return jnp.dot(a.astype(jnp.bfloat16), b.astype(jnp.bfloat16),
                   preferred_element_type=jnp.float32)


def _mlp_body(x_ref, ln_g, ln_b, w1, b1, g1, bb1, w2, b2, g2, bb2,
              w3, b3, g3, bb3, w4, b4, out_ref):
    x = _ln(x_ref[...], ln_g[...], ln_b[...])
    x = _leaky(_ln(_bdot(x, w1[...]) + b1[...], g1[...], bb1[...]))
    x = _leaky(_ln(_bdot(x, w2[...]) + b2[...], g2[...], bb2[...]))
    h = _leaky(_ln(_bdot(x, w3[...]) + b3[...], g3[...], bb3[...]))
    out_ref[...] = _bdot(h, w4[...]) + b4[...]


def _mlp_tc(pooled, ln_g, ln_b, w1, b1, g1, bb1, w2, b2, g2, bb2,
            w3, b3, g3, bb3, w4, b4):
    n, c = pooled.shape
    blk = 2048
    grid = n // blk

    def full(a):
        return pl.BlockSpec(a.shape, lambda i: (0,) * a.ndim)

    wargs = (ln_g, ln_b, w1, b1, g1, bb1, w2, b2, g2, bb2, w3, b3, g3, bb3, w4, b4)
    return pl.pallas_call(
        _mlp_body,
        grid=(grid,),
        in_specs=[pl.BlockSpec((blk, c), lambda i: (i, 0))] + [full(a) for a in wargs],
        out_specs=pl.BlockSpec((blk, 1), lambda i: (i, 0)),
        out_shape=jax.ShapeDtypeStruct((n, 1), jnp.float32),
    )(pooled, *wargs)


def kernel(feats, depths, proj, ln_g, ln_b, r_w1, r_b1, r_g1, r_bb1,
           r_w2, r_b2, r_g2, r_bb2, o_w1, o_b1, o_g1, o_bb1, o_w2, o_b2):
    H, W = feats.shape[2], feats.shape[3]
    C = feats.shape[1]
    tab = jnp.transpose(feats[0], (1, 2, 0)).reshape(H * W, C)
    dflat = depths.reshape(-1)
    projp = jnp.pad(proj.reshape(-1), (0, 4))
    pooled = _pool_sc(tab, dflat, projp, (H, W))
    return _mlp_tc(pooled, ln_g, ln_b, r_w1, r_b1, r_g1, r_bb1,
                   r_w2, r_b2, r_g2, r_bb2, o_w1, o_b1, o_g1, o_bb1, o_w2, o_b2)


# speedup vs baseline: 1.3378x; 1.3378x over previous
import functools

import jax
import jax.numpy as jnp
import numpy as np
from jax import lax
from jax.experimental import pallas as pl
from jax.experimental.pallas import tpu as pltpu
from jax.experimental.pallas import tpu_sc as plsc

VS = 0.4
ORG = (0.0, -25.6, -2.0)
IMH, IMW = 370, 1220
DX, DY, DZ = 128, 128, 16
NVOX = DX * DY * DZ
B = 128


def _f2i_floor(x):
    t = x.astype(jnp.int32)
    return t - jnp.where(t.astype(jnp.float32) > x, 1, 0)


def _sanitize(x):
    x = jnp.where(x != x, 0.0, x)
    return jnp.clip(x, -2.0e9, 2.0e9)


def _bf16r(x):
    y = lax.bitcast_convert_type(x, jnp.uint32)
    r = (y + jnp.uint32(0x7FFF) + ((y >> 16) & jnp.uint32(1))) & jnp.uint32(0xFFFF0000)
    return lax.bitcast_convert_type(r, jnp.float32)


def _pool_sc(tabi, dflat, projp, hw, c_total):
    H, W = hw
    C = c_total
    n_d = dflat.shape[0]
    mesh = plsc.VectorSubcoreMesh(core_axis_name="c", subcore_axis_name="s")
    info = plsc.get_sparse_core_info()
    nw = info.num_cores * info.num_subcores
    per_w = NVOX // nw
    n_chunks = per_w // B
    n_pairs = n_chunks // 2

    idx_t = pltpu.VMEM((B,), jnp.int32)
    wgt_t = pltpu.VMEM((B,), jnp.float32)
    row_t = pltpu.VMEM((B, C // 2), jnp.int32)

    @functools.partial(
        pl.kernel,
        mesh=mesh,
        out_type=jax.ShapeDtypeStruct((NVOX, C), jnp.float32),
        scratch_types=(
            [pltpu.VMEM((n_d,), jnp.float32), pltpu.VMEM((16,), jnp.float32)]
            + [idx_t] * 8 + [wgt_t] * 8 + [row_t] * 8
            + [pltpu.VMEM((B, C), jnp.float32),
               pltpu.SemaphoreType.DMA, pltpu.SemaphoreType.DMA]
        ),
        compiler_params=pltpu.CompilerParams(needs_layout_passes=False,
                                             use_tc_tiling_on_sc=False),
    )
    def pool(tab_h, dep_h, proj_h, out_h, dep_v, proj_v,
             iA0, iA1, iA2, iA3, iB0, iB1, iB2, iB3,
             wA0, wA1, wA2, wA3, wB0, wB1, wB2, wB3,
             rA0, rA1, rA2, rA3, rB0, rB1, rB2, rB3,
             out_v, semA, semB):
        iA = [iA0, iA1, iA2, iA3]
        iB = [iB0, iB1, iB2, iB3]
        wA = [wA0, wA1, wA2, wA3]
        wB = [wB0, wB1, wB2, wB3]
        rA = [rA0, rA1, rA2, rA3]
        rB = [rB0, rB1, rB2, rB3]
        wid = lax.axis_index("s") * info.num_cores + lax.axis_index("c")
        pltpu.sync_copy(dep_h, dep_v)
        pltpu.sync_copy(proj_h, proj_v)
        pvec = _bf16r(proj_v[...])
        p = [pvec[c] for c in range(12)]

        wspan = jnp.float32(W - 1)
        hspan = jnp.float32(H - 1)

        def geom_chunk(t, iS, wS):
            base = wid * per_w + t * B

            def geom(g, bs):
                nvec = bs + g * 16 + lax.iota(jnp.int32, 16)
                fi = (nvec >> 11).astype(jnp.float32)
                fj = ((nvec >> 4) & (DY - 1)).astype(jnp.float32)
                fk = (nvec & (DZ - 1)).astype(jnp.float32)
                fx = _bf16r(fi * VS + (0.5 * VS + ORG[0]))
                fy = _bf16r(fj * VS + (0.5 * VS + ORG[1]))
                fz = _bf16r(fk * VS + (0.5 * VS + ORG[2]))
                ix_ = ((p[0] * fx + p[1] * fy) + p[2] * fz) + p[3]
                iy_ = ((p[4] * fx + p[5] * fy) + p[6] * fz) + p[7]
                iz_ = ((p[8] * fx + p[9] * fy) + p[10] * fz) + p[11]
                u = ix_ / iz_
                v = iy_ / iz_
                gx = (2.0 * u) / jnp.float32(IMW - 1) - 1.0
                gy = (2.0 * v) / jnp.float32(IMH - 1) - 1.0
                big = jnp.float32(3.4028235e38)
                gx = jnp.where(jnp.abs(gx) > big, -2.0, gx)
                gy = jnp.where(jnp.abs(gy) > big, -2.0, gy)
                mask = (jnp.abs(gx) <= 1.0) & (jnp.abs(gy) <= 1.0) & (iz_ > 0.0)

                ixn = _f2i_floor(_sanitize(((gx + 1.0) * W - 1.0) / 2.0 + 0.5))
                iyn = _f2i_floor(_sanitize(((gy + 1.0) * H - 1.0) / 2.0 + 0.5))
                dvalid = (ixn >= 0) & (ixn < W) & (iyn >= 0) & (iyn < H)
                didx = jnp.clip(iyn, 0, H - 1) * W + jnp.clip(ixn, 0, W - 1)
                dep = plsc.load_gather(dep_v, [didx])
                dep = jnp.where(dvalid, dep, 0.0)
                diff = iz_ - dep
                prob = jnp.exp(-(diff * diff) / jnp.float32((2 * VS) ** 2))
                scale = jnp.where(mask, prob * 100.0, 0.0)

                def refl(g_, span):
                    x = jnp.abs(_sanitize((g_ + 1.0) / 2.0 * span))
                    x = lax.rem(x, 2.0 * span)
                    return jnp.where(x > span, 2.0 * span - x, x)

                fxc = refl(gx, wspan)
                fyc = refl(gy, hspan)
                x0 = _f2i_floor(fxc)
                y0 = _f2i_floor(fyc)
                wx1 = fxc - x0.astype(jnp.float32)
                wy1 = fyc - y0.astype(jnp.float32)
                x0i = jnp.clip(x0, 0, W - 1)
                x1i = jnp.minimum(x0i + 1, W - 1)
                y0i = jnp.clip(y0, 0, H - 1)
                y1i = jnp.minimum(y0i + 1, H - 1)
                wx0 = 1.0 - wx1
                wy0 = 1.0 - wy1
                sl = pl.ds(g * 16, 16)
                iS[0][sl] = y0i * W + x0i
                iS[1][sl] = y0i * W + x1i
                iS[2][sl] = y1i * W + x0i
                iS[3][sl] = y1i * W + x1i
                wS[0][sl] = wy0 * wx0 * scale
                wS[1][sl] = wy0 * wx1 * scale
                wS[2][sl] = wy1 * wx0 * scale
                wS[3][sl] = wy1 * wx1 * scale
                return bs

            lax.fori_loop(0, B // 16, geom, base, unroll=2)

        def fire(iS, rS, sem):
            for c in range(4):
                pltpu.async_copy(tab_h.at[iS[c]], rS[c], sem)

        def drain(iS, rS, sem):
            for c in range(4):
                pltpu.make_async_copy(tab_h.at[iS[c]], rS[c], sem).wait()

        def blend(rS, wS):
            def bb(g, _):
                sl = pl.ds(g * 16, 16)
                wv = [wS[c][sl] for c in range(4)]
                for li in range(16):
                    vv = g * 16 + li
                    ws = [wv[c][li] for c in range(4)]
                    for cc in range(C // 32):
                        s16 = pl.ds(cc * 16, 16)
                        pr = [plsc.unpack(plsc.bitcast(rS[c][vv, s16], jnp.bfloat16),
                                          format=plsc.PackFormat.INTERLEAVED)
                              for c in range(4)]
                        ea = (ws[0] * pr[0][0] + ws[1] * pr[1][0]
                              + ws[2] * pr[2][0] + ws[3] * pr[3][0])
                        eb = (ws[0] * pr[0][1] + ws[1] * pr[1][1]
                              + ws[2] * pr[2][1] + ws[3] * pr[3][1])
                        out_v[vv, pl.ds(cc * 32, 16)] = ea
                        out_v[vv, pl.ds(cc * 32 + 16, 16)] = eb
                return 0

            lax.fori_loop(0, B // 16, bb, 0)

        geom_chunk(0, iA, wA)
        fire(iA, rA, semA)

        def pair(t2, _):
            t = 2 * t2
            geom_chunk(t + 1, iB, wB)
            fire(iB, rB, semB)
            drain(iA, rA, semA)
            blend(rA, wA)
            pltpu.sync_copy(out_v, out_h.at[pl.ds(wid * per_w + t * B, B)])

            @pl.when(t2 + 1 < n_pairs)
            def _():
                geom_chunk(t + 2, iA, wA)
                fire(iA, rA, semA)

            drain(iB, rB, semB)
            blend(rB, wB)
            pltpu.sync_copy(out_v, out_h.at[pl.ds(wid * per_w + (t + 1) * B, B)])
            return 0

        lax.fori_loop(0, n_pairs, pair, 0)

    return pool(tabi, dflat, projp)


def _ln(x, g, b):
    m = jnp.mean(x, -1, keepdims=True)
    v = jnp.mean((x - m) ** 2, -1, keepdims=True)
    return (x - m) / jnp.sqrt(v + 1e-5) * g + b


def _leaky(x):
    return jnp.maximum(x, 0.1 * x)


def _bdot(a, b):
    return jnp.dot(a.astype(jnp.bfloat16), b.astype(jnp.bfloat16),
                   preferred_element_type=jnp.float32)


def _mlp_body(x_ref, ln_g, ln_b, w1, b1, g1, bb1, w2, b2, g2, bb2,
              w3, b3, g3, bb3, w4, b4, out_ref):
    x = _ln(x_ref[...], ln_g[...], ln_b[...])
    x = _leaky(_ln(_bdot(x, w1[...]) + b1[...], g1[...], bb1[...]))
    x = _leaky(_ln(_bdot(x, w2[...]) + b2[...], g2[...], bb2[...]))
    h = _leaky(_ln(_bdot(x, w3[...]) + b3[...], g3[...], bb3[...]))
    out_ref[...] = _bdot(h, w4[...]) + b4[...]


def _mlp_tc(pooled, ln_g, ln_b, w1, b1, g1, bb1, w2, b2, g2, bb2,
            w3, b3, g3, bb3, w4, b4):
    n, c = pooled.shape
    blk = 2048
    grid = n // blk

    def full(a):
        return pl.BlockSpec(a.shape, lambda i: (0,) * a.ndim)

    wargs = (ln_g, ln_b, w1, b1, g1, bb1, w2, b2, g2, bb2, w3, b3, g3, bb3, w4, b4)
    return pl.pallas_call(
        _mlp_body,
        grid=(grid,),
        in_specs=[pl.BlockSpec((blk, c), lambda i: (i, 0))] + [full(a) for a in wargs],
        out_specs=pl.BlockSpec((blk, 1), lambda i: (i, 0)),
        out_shape=jax.ShapeDtypeStruct((n, 1), jnp.float32),
    )(pooled, *wargs)


def _chan_perm(c):
    perm = np.empty(c, np.int32)
    for c0 in range(0, c, 32):
        for k in range(16):
            perm[c0 + 2 * k] = c0 + k
            perm[c0 + 2 * k + 1] = c0 + 16 + k
    return perm


def kernel(feats, depths, proj, ln_g, ln_b, r_w1, r_b1, r_g1, r_bb1,
           r_w2, r_b2, r_g2, r_bb2, o_w1, o_b1, o_g1, o_bb1, o_w2, o_b2):
    H, W = feats.shape[2], feats.shape[3]
    C = feats.shape[1]
    tab = jnp.transpose(feats[0], (1, 2, 0)).reshape(H * W, C)
    tabi = tab[:, _chan_perm(C)].astype(jnp.bfloat16)
    tab32 = lax.bitcast_convert_type(tabi.reshape(H * W, C // 2, 2), jnp.int32)
    dflat = depths.reshape(-1)
    projp = jnp.pad(proj.reshape(-1), (0, 4))
    pooled = _pool_sc(tab32, dflat, projp, (H, W), C)
    return _mlp_tc(pooled, ln_g, ln_b, r_w1, r_b1, r_g1, r_bb1,
                   r_w2, r_b2, r_g2, r_bb2, o_w1, o_b1, o_g1, o_bb1, o_w2, o_b2)


# speedup vs baseline: 1.6943x; 1.2665x over previous
import functools

import jax
import jax.numpy as jnp
import numpy as np
from jax import lax
from jax.experimental import pallas as pl
from jax.experimental.pallas import tpu as pltpu
from jax.experimental.pallas import tpu_sc as plsc

VS = 0.4
ORG = (0.0, -25.6, -2.0)
IMH, IMW = 370, 1220
DX, DY, DZ = 128, 128, 16
NVOX = DX * DY * DZ
B = 128


def _f2i_floor(x):
    t = x.astype(jnp.int32)
    return t - jnp.where(t.astype(jnp.float32) > x, 1, 0)


def _sanitize(x):
    x = jnp.where(x != x, 0.0, x)
    return jnp.clip(x, -2.0e9, 2.0e9)


def _bf16r(x):
    y = lax.bitcast_convert_type(x, jnp.uint32)
    r = (y + jnp.uint32(0x7FFF) + ((y >> 16) & jnp.uint32(1))) & jnp.uint32(0xFFFF0000)
    return lax.bitcast_convert_type(r, jnp.float32)


def _pool_sc(tabi, dflat, projp, hw, c_total, n_base, n_len):
    H, W = hw
    C = c_total
    n_d = dflat.shape[0]
    mesh = plsc.VectorSubcoreMesh(core_axis_name="c", subcore_axis_name="s")
    info = plsc.get_sparse_core_info()
    nw = info.num_cores * info.num_subcores
    per_w = n_len // nw
    n_chunks = per_w // B
    n_pairs = n_chunks // 2

    idx_t = pltpu.VMEM((B,), jnp.int32)
    wgt_t = pltpu.VMEM((B,), jnp.float32)
    row_t = pltpu.VMEM((B, C // 2), jnp.int32)

    @functools.partial(
        pl.kernel,
        mesh=mesh,
        out_type=jax.ShapeDtypeStruct((n_len, C), jnp.float32),
        scratch_types=(
            [pltpu.VMEM((n_d,), jnp.float32), pltpu.VMEM((16,), jnp.float32)]
            + [idx_t] * 8 + [wgt_t] * 8 + [row_t] * 8
            + [pltpu.VMEM((B, C), jnp.float32),
               pltpu.SemaphoreType.DMA, pltpu.SemaphoreType.DMA]
        ),
        compiler_params=pltpu.CompilerParams(needs_layout_passes=False,
                                             use_tc_tiling_on_sc=False),
    )
    def pool(tab_h, dep_h, proj_h, out_h, dep_v, proj_v,
             iA0, iA1, iA2, iA3, iB0, iB1, iB2, iB3,
             wA0, wA1, wA2, wA3, wB0, wB1, wB2, wB3,
             rA0, rA1, rA2, rA3, rB0, rB1, rB2, rB3,
             out_v, semA, semB):
        iA = [iA0, iA1, iA2, iA3]
        iB = [iB0, iB1, iB2, iB3]
        wA = [wA0, wA1, wA2, wA3]
        wB = [wB0, wB1, wB2, wB3]
        rA = [rA0, rA1, rA2, rA3]
        rB = [rB0, rB1, rB2, rB3]
        wid = lax.axis_index("s") * info.num_cores + lax.axis_index("c")
        pltpu.sync_copy(dep_h, dep_v)
        pltpu.sync_copy(proj_h, proj_v)
        pvec = _bf16r(proj_v[...])
        p = [pvec[c] for c in range(12)]

        wspan = jnp.float32(W - 1)
        hspan = jnp.float32(H - 1)

        def geom_chunk(t, iS, wS):
            base = n_base + wid * per_w + t * B

            def geom(g, bs):
                nvec = bs + g * 16 + lax.iota(jnp.int32, 16)
                fi = (nvec >> 11).astype(jnp.float32)
                fj = ((nvec >> 4) & (DY - 1)).astype(jnp.float32)
                fk = (nvec & (DZ - 1)).astype(jnp.float32)
                fx = _bf16r(fi * VS + (0.5 * VS + ORG[0]))
                fy = _bf16r(fj * VS + (0.5 * VS + ORG[1]))
                fz = _bf16r(fk * VS + (0.5 * VS + ORG[2]))
                ix_ = ((p[0] * fx + p[1] * fy) + p[2] * fz) + p[3]
                iy_ = ((p[4] * fx + p[5] * fy) + p[6] * fz) + p[7]
                iz_ = ((p[8] * fx + p[9] * fy) + p[10] * fz) + p[11]
                u = ix_ / iz_
                v = iy_ / iz_
                gx = (2.0 * u) / jnp.float32(IMW - 1) - 1.0
                gy = (2.0 * v) / jnp.float32(IMH - 1) - 1.0
                big = jnp.float32(3.4028235e38)
                gx = jnp.where(jnp.abs(gx) > big, -2.0, gx)
                gy = jnp.where(jnp.abs(gy) > big, -2.0, gy)
                mask = (jnp.abs(gx) <= 1.0) & (jnp.abs(gy) <= 1.0) & (iz_ > 0.0)

                ixn = _f2i_floor(_sanitize(((gx + 1.0) * W - 1.0) / 2.0 + 0.5))
                iyn = _f2i_floor(_sanitize(((gy + 1.0) * H - 1.0) / 2.0 + 0.5))
                dvalid = (ixn >= 0) & (ixn < W) & (iyn >= 0) & (iyn < H)
                didx = jnp.clip(iyn, 0, H - 1) * W + jnp.clip(ixn, 0, W - 1)
                dep = plsc.load_gather(dep_v, [didx])
                dep = jnp.where(dvalid, dep, 0.0)
                diff = iz_ - dep
                prob = jnp.exp(-(diff * diff) / jnp.float32((2 * VS) ** 2))
                scale = jnp.where(mask, prob * 100.0, 0.0)

                def refl(g_, span):
                    x = jnp.abs(_sanitize((g_ + 1.0) / 2.0 * span))
                    x = lax.rem(x, 2.0 * span)
                    return jnp.where(x > span, 2.0 * span - x, x)

                fxc = refl(gx, wspan)
                fyc = refl(gy, hspan)
                x0 = _f2i_floor(fxc)
                y0 = _f2i_floor(fyc)
                wx1 = fxc - x0.astype(jnp.float32)
                wy1 = fyc - y0.astype(jnp.float32)
                x0i = jnp.clip(x0, 0, W - 1)
                x1i = jnp.minimum(x0i + 1, W - 1)
                y0i = jnp.clip(y0, 0, H - 1)
                y1i = jnp.minimum(y0i + 1, H - 1)
                wx0 = 1.0 - wx1
                wy0 = 1.0 - wy1
                sl = pl.ds(g * 16, 16)
                iS[0][sl] = y0i * W + x0i
                iS[1][sl] = y0i * W + x1i
                iS[2][sl] = y1i * W + x0i
                iS[3][sl] = y1i * W + x1i
                wS[0][sl] = wy0 * wx0 * scale
                wS[1][sl] = wy0 * wx1 * scale
                wS[2][sl] = wy1 * wx0 * scale
                wS[3][sl] = wy1 * wx1 * scale
                return bs

            lax.fori_loop(0, B // 16, geom, base, unroll=2)

        def fire(iS, rS, sem):
            for c in range(4):
                pltpu.async_copy(tab_h.at[iS[c]], rS[c], sem)

        def drain(iS, rS, sem):
            for c in range(4):
                pltpu.make_async_copy(tab_h.at[iS[c]], rS[c], sem).wait()

        def blend(rS, wS):
            def bb(g, _):
                sl = pl.ds(g * 16, 16)
                wv = [wS[c][sl] for c in range(4)]
                for li in range(16):
                    vv = g * 16 + li
                    ws = [wv[c][li] for c in range(4)]
                    for cc in range(C // 32):
                        s16 = pl.ds(cc * 16, 16)
                        pr = [plsc.unpack(plsc.bitcast(rS[c][vv, s16], jnp.bfloat16),
                                          format=plsc.PackFormat.INTERLEAVED)
                              for c in range(4)]
                        ea = (ws[0] * pr[0][0] + ws[1] * pr[1][0]
                              + ws[2] * pr[2][0] + ws[3] * pr[3][0])
                        eb = (ws[0] * pr[0][1] + ws[1] * pr[1][1]
                              + ws[2] * pr[2][1] + ws[3] * pr[3][1])
                        out_v[vv, pl.ds(cc * 32, 16)] = ea
                        out_v[vv, pl.ds(cc * 32 + 16, 16)] = eb
                return 0

            lax.fori_loop(0, B // 16, bb, 0)

        geom_chunk(0, iA, wA)
        fire(iA, rA, semA)

        def pair(t2, _):
            t = 2 * t2
            geom_chunk(t + 1, iB, wB)
            fire(iB, rB, semB)
            drain(iA, rA, semA)
            blend(rA, wA)
            pltpu.sync_copy(out_v, out_h.at[pl.ds(wid * per_w + t * B, B)])

            @pl.when(t2 + 1 < n_pairs)
            def _():
                geom_chunk(t + 2, iA, wA)
                fire(iA, rA, semA)

            drain(iB, rB, semB)
            blend(rB, wB)
            pltpu.sync_copy(out_v, out_h.at[pl.ds(wid * per_w + (t + 1) * B, B)])
            return 0

        lax.fori_loop(0, n_pairs, pair, 0)

    return pool(tabi, dflat, projp)


def _ln(x, g, b):
    m = jnp.mean(x, -1, keepdims=True)
    v = jnp.mean((x - m) ** 2, -1, keepdims=True)
    return (x - m) / jnp.sqrt(v + 1e-5) * g + b


def _leaky(x):
    return jnp.maximum(x, 0.1 * x)


def _bdot(a, b):
    return jnp.dot(a.astype(jnp.bfloat16), b.astype(jnp.bfloat16),
                   preferred_element_type=jnp.float32)


def _mlp_body(x_ref, ln_g, ln_b, w1, b1, g1, bb1, w2, b2, g2, bb2,
              w3, b3, g3, bb3, w4, b4, out_ref):
    x = _ln(x_ref[...], ln_g[...], ln_b[...])
    x = _leaky(_ln(_bdot(x, w1[...]) + b1[...], g1[...], bb1[...]))
    x = _leaky(_ln(_bdot(x, w2[...]) + b2[...], g2[...], bb2[...]))
    h = _leaky(_ln(_bdot(x, w3[...]) + b3[...], g3[...], bb3[...]))
    out_ref[...] = _bdot(h, w4[...]) + b4[...]


def _mlp_tc(pooled, ln_g, ln_b, w1, b1, g1, bb1, w2, b2, g2, bb2,
            w3, b3, g3, bb3, w4, b4):
    n, c = pooled.shape
    blk = 2048
    grid = n // blk

    def full(a):
        return pl.BlockSpec(a.shape, lambda i: (0,) * a.ndim)

    wargs = (ln_g, ln_b, w1, b1, g1, bb1, w2, b2, g2, bb2, w3, b3, g3, bb3, w4, b4)
    return pl.pallas_call(
        _mlp_body,
        grid=(grid,),
        in_specs=[pl.BlockSpec((blk, c), lambda i: (i, 0))] + [full(a) for a in wargs],
        out_specs=pl.BlockSpec((blk, 1), lambda i: (i, 0)),
        out_shape=jax.ShapeDtypeStruct((n, 1), jnp.float32),
    )(pooled, *wargs)


def _chan_perm(c):
    perm = np.empty(c, np.int32)
    for c0 in range(0, c, 32):
        for k in range(16):
            perm[c0 + 2 * k] = c0 + k
            perm[c0 + 2 * k + 1] = c0 + 16 + k
    return perm


def kernel(feats, depths, proj, ln_g, ln_b, r_w1, r_b1, r_g1, r_bb1,
           r_w2, r_b2, r_g2, r_bb2, o_w1, o_b1, o_g1, o_bb1, o_w2, o_b2):
    H, W = feats.shape[2], feats.shape[3]
    C = feats.shape[1]
    tab = jnp.transpose(feats[0], (1, 2, 0)).reshape(H * W, C)
    tabi = tab[:, _chan_perm(C)].astype(jnp.bfloat16)
    tab32 = lax.bitcast_convert_type(tabi.reshape(H * W, C // 2, 2), jnp.int32)
    dflat = depths.reshape(-1)
    projp = jnp.pad(proj.reshape(-1), (0, 4))
    n_slices = 4
    slen = NVOX // n_slices
    outs = []
    for s in range(n_slices):
        ps = _pool_sc(tab32, dflat, projp, (H, W), C, s * slen, slen)
        outs.append(_mlp_tc(ps, ln_g, ln_b, r_w1, r_b1, r_g1, r_bb1,
                            r_w2, r_b2, r_g2, r_bb2,
                            o_w1, o_b1, o_g1, o_bb1, o_w2, o_b2))
    return jnp.concatenate(outs, axis=0)
